# baseline (speedup 1.0000x reference)
import functools

import jax
import jax.numpy as jnp
from jax import lax
from jax.experimental import pallas as pl
from jax.experimental.pallas import tpu as pltpu
from jax.experimental.pallas import tpu_sc as plsc

_B = 4096
_L = 200
_D = 64
_NC = 2
_NS = 16
_NW = _NC * _NS
_ROWS = _B // _NW
_CHUNK = 100
_NCHUNK = _L // _CHUNK
_NV = _D // 16


def _sc_pool_body(x_hbm, tab_hbm, out_hbm,
                  idx_v, rows0, rows1, pooled_v, sem0, sem1):
    wid = lax.axis_index("s") * _NC + lax.axis_index("c")
    base = wid * _ROWS
    pltpu.sync_copy(x_hbm.at[pl.ds(base * _NCHUNK, _ROWS * _NCHUNK)], idx_v)

    def copies(r, rows_v, sem):
        return [
            pltpu.make_async_copy(
                tab_hbm.at[idx_v.at[r * _NCHUNK + c]],
                rows_v.at[pl.ds(c * _CHUNK, _CHUNK)],
                sem,
            )
            for c in range(_NCHUNK)
        ]

    def start(r, rows_v, sem):
        for cp in copies(r, rows_v, sem):
            cp.start()

    def wait(r, rows_v, sem):
        for cp in copies(r, rows_v, sem):
            cp.wait()

    inv_len = jnp.full((16,), 1.0 / _L, dtype=jnp.float32)

    def reduce_row(rows_v, r):
        def body(j, accs):
            return tuple(a + rows_v[j, pl.ds(v * 16, 16)]
                         for v, a in enumerate(accs))
        accs = lax.fori_loop(
            0, _L, body,
            tuple(jnp.zeros((16,), jnp.float32) for _ in range(_NV)),
            unroll=8)
        for v, a in enumerate(accs):
            pooled_v[r, pl.ds(v * 16, 16)] = a * inv_len

    start(0, rows0, sem0)

    def outer(k, carry):
        r0 = 2 * k
        wait(r0, rows0, sem0)
        start(r0 + 1, rows1, sem1)
        reduce_row(rows0, r0)
        wait(r0 + 1, rows1, sem1)

        @pl.when(r0 + 2 < _ROWS)
        def _():
            start(r0 + 2, rows0, sem0)

        reduce_row(rows1, r0 + 1)
        return carry

    lax.fori_loop(0, _ROWS // 2, outer, 0)
    pltpu.sync_copy(pooled_v, out_hbm.at[pl.ds(base, _ROWS)])


@functools.partial(jax.jit, static_argnums=())
def _sc_pool(x2, table):
    mesh = plsc.VectorSubcoreMesh(core_axis_name="c", subcore_axis_name="s")
    return pl.kernel(
        _sc_pool_body,
        out_type=jax.ShapeDtypeStruct((_B, _D), jnp.float32),
        mesh=mesh,
        scratch_types=[
            pltpu.VMEM((_ROWS * _NCHUNK, _CHUNK), jnp.int32),
            pltpu.VMEM((_L, _D), jnp.float32),
            pltpu.VMEM((_L, _D), jnp.float32),
            pltpu.VMEM((_ROWS, _D), jnp.float32),
            pltpu.SemaphoreType.DMA,
            pltpu.SemaphoreType.DMA,
        ],
        compiler_params=pltpu.CompilerParams(use_tc_tiling_on_sc=False),
    )(x2, table)


def _mlp_body(p_ref, w1_ref, b1_ref, w2_ref, b2_ref, o_ref):
    h = jnp.dot(p_ref[...], w1_ref[...], preferred_element_type=jnp.float32)
    h = jnp.maximum(h + b1_ref[...], 0.0)
    o = jnp.dot(h, w2_ref[...], preferred_element_type=jnp.float32)
    o = o + b2_ref[...]
    m = jnp.max(o, axis=1, keepdims=True)
    lse = jnp.log(jnp.sum(jnp.exp(o - m), axis=1, keepdims=True)) + m
    o_ref[...] = o - lse


def _mlp(pooled, W1, b1, W2, b2):
    return pl.pallas_call(
        _mlp_body,
        out_shape=jax.ShapeDtypeStruct((_B, 2), jnp.float32),
    )(pooled, W1, b1, W2, b2)


def kernel(x, embedding_matrix, W1, b1, W2, b2):
    x2 = x.astype(jnp.int32).reshape(_B * _NCHUNK, _CHUNK)
    pooled = _sc_pool(x2, embedding_matrix)
    return _mlp(pooled, W1, b1.reshape(1, -1), W2, b2.reshape(1, -1))
